# baseline (speedup 1.0000x reference)
import functools

import jax
import jax.numpy as jnp
from jax.experimental import pallas as pl
from jax.experimental.pallas import tpu as pltpu

EPS = 1e-5


def _ln(x, g, b):
    mu = jnp.mean(x, axis=-1, keepdims=True)
    var = jnp.mean((x - mu) ** 2, axis=-1, keepdims=True)
    return (x - mu) * jax.lax.rsqrt(var + EPS) * g + b


def _elu(x):
    return jnp.where(x > 0, x, jnp.exp(x) - 1.0)


def _node_kernel(msa_ref, seq_ref, gm_ref, bm_ref, Wq_ref, bq_ref, WkT_ref,
                 Wnm_ref, Wns_ref, bn_ref, gn_ref, bnn_ref, node_ref, *, scale):
    x = msa_ref[...]
    xn = _ln(x, gm_ref[...], bm_ref[...])
    q = (jnp.dot(xn[0], Wq_ref[...], preferred_element_type=jnp.float32)
         + bq_ref[...]) * scale
    qw = jnp.dot(q, WkT_ref[...], preferred_element_type=jnp.float32)
    logits = jnp.sum(xn * qw[None, :, :], axis=-1)
    mx = jnp.max(logits, axis=0, keepdims=True)
    e = jnp.exp(logits - mx)
    att = e / jnp.sum(e, axis=0, keepdims=True)
    ws = jnp.sum(xn * att[:, :, None], axis=0)
    pre = (jnp.dot(ws, Wnm_ref[...], preferred_element_type=jnp.float32)
           + jnp.dot(seq_ref[...], Wns_ref[...], preferred_element_type=jnp.float32)
           + bn_ref[...])
    node_ref[...] = _ln(_elu(pre), gn_ref[...], bnn_ref[...])


def _mask_kernel(cac_ref, car_ref, aac_ref, aar_ref, mask_ref, *, L, K, kmin):
    dx = [car_ref[c:c + 1, :] - cac_ref[:, c:c + 1] for c in range(3)]
    pd2 = dx[0] * dx[0] + dx[1] * dx[1] + dx[2] * dx[2]
    ri = jax.lax.broadcasted_iota(jnp.int32, (L, L), 0)
    ci = jax.lax.broadcasted_iota(jnp.int32, (L, L), 1)
    diag = ri == ci
    pdist = jnp.sqrt(pd2 + 1e-12) + jnp.where(diag, 1000.0, 0.0)
    bits = jax.lax.bitcast_convert_type(pdist, jnp.int32)

    def body(_, carry):
        lo, hi = carry
        mid = lo + jax.lax.shift_right_logical(hi - lo, 1)
        cnt = jnp.sum((bits <= mid).astype(jnp.int32), axis=1, keepdims=True)
        ge = cnt >= K
        return jnp.where(ge, lo, mid), jnp.where(ge, mid, hi)

    lo0 = jnp.full((L, 1), -1, jnp.int32)
    hi0 = jnp.full((L, 1), 0x7F7FFFFF, jnp.int32)
    _, t = jax.lax.fori_loop(0, 31, body, (lo0, hi0))

    lt = bits < t
    eqt = bits == t
    c_lt = jnp.sum(lt.astype(jnp.int32), axis=1, keepdims=True)
    needed = K - c_lt
    def body2(_, carry):
        lo, hi = carry
        mid = lo + jax.lax.shift_right_logical(hi - lo, 1)
        cnt = jnp.sum(jnp.logical_and(eqt, ci <= mid).astype(jnp.int32),
                      axis=1, keepdims=True)
        ge = cnt >= needed
        return jnp.where(ge, lo, mid), jnp.where(ge, mid, hi)

    jlo0 = jnp.full((L, 1), -1, jnp.int32)
    jhi0 = jnp.full((L, 1), L - 1, jnp.int32)
    _, jt = jax.lax.fori_loop(0, 10, body2, (jlo0, jhi0))
    knn = jnp.logical_or(lt, jnp.logical_and(eqt, ci <= jt))
    aa_d = jnp.abs(aac_ref[...] - aar_ref[...])
    band = jnp.logical_and(aa_d < kmin, jnp.logical_not(diag))
    mask_ref[...] = jnp.where(jnp.logical_or(knn, band), 1.0, 0.0)


def _mp_kernel(pair_ref, mask_ref, node_i_ref, node_j_ref, cai_ref, caj_ref,
               xyz9_ref, gp_ref, bp_ref, We_ref, be_ref, ge_ref, bee_ref,
               Wna_ref, Wea_ref, wda_ref, ba_ref, Wself_ref, gs_ref, bs_ref,
               state_ref, xyzo_ref, agg_acc, disp_acc, *, nf):
    i = pl.program_id(1)
    ni = pl.num_programs(1)
    Ib, Jb, Dp = pair_ref.shape
    x = pair_ref[...].reshape(Ib * Jb, Dp)
    xn = _ln(x, gp_ref[...], bp_ref[...])
    e0 = _elu(jnp.dot(xn, We_ref[...], preferred_element_type=jnp.float32)
              + be_ref[...])
    edge = _ln(e0, ge_ref[...], bee_ref[...])
    pre = jnp.dot(edge, Wea_ref[...], preferred_element_type=jnp.float32)
    pre = pre.reshape(Ib, Jb, nf)
    nterm = jnp.dot(node_i_ref[...], Wna_ref[...], preferred_element_type=jnp.float32)
    dx = [caj_ref[c:c + 1, :] - cai_ref[:, c:c + 1] for c in range(3)]
    dist = jnp.sqrt(dx[0] ** 2 + dx[1] ** 2 + dx[2] ** 2)
    pre = (pre + nterm[:, None, :] + dist[:, :, None] * wda_ref[...][None]
           + ba_ref[...][None])
    w = mask_ref[...]
    m = _elu(pre[:, :, :32]) * w[:, :, None]
    aggc = jnp.sum(m, axis=0)
    mcoef = pre[:, :, 32:] * w[:, :, None]
    cols = []
    for a in range(3):
        for c in range(3):
            cols.append(jnp.sum(mcoef[:, :, a] * dx[c], axis=0).reshape(Jb, 1))
    dispc = jnp.concatenate(cols, axis=1)

    @pl.when(i == 0)
    def _():
        agg_acc[...] = jnp.zeros_like(agg_acc)
        disp_acc[...] = jnp.zeros_like(disp_acc)

    agg_acc[...] += aggc
    disp_acc[...] += dispc

    @pl.when(i == ni - 1)
    def _():
        agg = agg_acc[...] + jnp.dot(node_j_ref[...], Wself_ref[...],
                                     preferred_element_type=jnp.float32)
        state_ref[...] = _ln(_elu(agg), gs_ref[...], bs_ref[...])
        d = disp_acc[...]
        xin = xyz9_ref[...]
        ca_new = xin[:, 3:6] + d[:, 3:6]
        xyzo_ref[...] = jnp.concatenate(
            [ca_new + d[:, 0:3], ca_new, ca_new + d[:, 6:9]], axis=1)


def _full(shape):
    return pl.BlockSpec(shape, lambda *args: (0,) * len(shape))


def kernel(xyz, msa, pair, seq_onehot, params, aa_idx, interpret=False):
    p = params
    B, L = xyz.shape[:2]
    N, Dm = msa.shape[1], msa.shape[3]
    Dp = pair.shape[3]
    Dn, Ds, NF = 64, 32, 35
    K, KMIN = 64, 9

    msa3 = msa[0]
    pair3 = pair[0]
    seq = seq_onehot[0]
    ca = xyz[0, :, 1, :]
    cac = jnp.pad(ca, ((0, 0), (0, 5)))
    car = cac.T
    aa = aa_idx[0].astype(jnp.int32)
    aac = aa.reshape(L, 1)
    aar = aa.reshape(1, L)
    xyz9 = xyz[0].reshape(L, 9)
    scale = float(Dm) ** -0.5

    gm = p['ln_msa_g'].reshape(1, 1, Dm)
    bm = p['ln_msa_b'].reshape(1, 1, Dm)
    bq = p['bq'].reshape(1, Dm)
    WkT = p['Wk'].T
    Wnm = p['W_n'][:Dm]
    Wns = p['W_n'][Dm:]
    bn = p['b_n'].reshape(1, Dn)
    gn = p['ln_node_g'].reshape(1, Dn)
    bnn = p['ln_node_b'].reshape(1, Dn)
    gp = p['ln_pair_g'].reshape(1, Dp)
    bp = p['ln_pair_b'].reshape(1, Dp)
    be = p['b_e'].reshape(1, Dn)
    ge = p['ln_edge_g'].reshape(1, Dn)
    bee = p['ln_edge_b'].reshape(1, Dn)
    W_all = jnp.concatenate([p['W_msg'], p['W_vec']], axis=1)
    Wna = W_all[:Dn]
    Wea = W_all[Dn:2 * Dn]
    wda = W_all[2 * Dn].reshape(1, NF)
    ba = jnp.concatenate([p['b_msg'], jnp.zeros((3,), jnp.float32)]).reshape(1, NF)
    gs = p['ln_state_g'].reshape(1, Ds)
    bs = p['ln_state_b'].reshape(1, Ds)

    LB = 64
    node = pl.pallas_call(
        functools.partial(_node_kernel, scale=scale),
        grid=(L // LB,),
        in_specs=[
            pl.BlockSpec((N, LB, Dm), lambda l: (0, l, 0)),
            pl.BlockSpec((LB, seq.shape[1]), lambda l: (l, 0)),
            _full(gm.shape), _full(bm.shape), _full(p['Wq'].shape),
            _full(bq.shape), _full(WkT.shape), _full(Wnm.shape),
            _full(Wns.shape), _full(bn.shape), _full(gn.shape),
            _full(bnn.shape),
        ],
        out_specs=pl.BlockSpec((LB, Dn), lambda l: (l, 0)),
        out_shape=jax.ShapeDtypeStruct((L, Dn), jnp.float32),
        interpret=interpret,
    )(msa3, seq, gm, bm, p['Wq'], bq, WkT, Wnm, Wns, bn, gn, bnn)

    mask = pl.pallas_call(
        functools.partial(_mask_kernel, L=L, K=K, kmin=KMIN),
        out_shape=jax.ShapeDtypeStruct((L, L), jnp.float32),
        interpret=interpret,
    )(cac, car, aac, aar)

    IB, JB = 64, 128
    grid = (L // JB, L // IB)
    state, xyzo = pl.pallas_call(
        functools.partial(_mp_kernel, nf=NF),
        grid=grid,
        in_specs=[
            pl.BlockSpec((IB, JB, Dp), lambda j, i: (i, j, 0)),
            pl.BlockSpec((IB, JB), lambda j, i: (i, j)),
            pl.BlockSpec((IB, Dn), lambda j, i: (i, 0)),
            pl.BlockSpec((JB, Dn), lambda j, i: (j, 0)),
            pl.BlockSpec((IB, 8), lambda j, i: (i, 0)),
            pl.BlockSpec((8, JB), lambda j, i: (0, j)),
            pl.BlockSpec((JB, 9), lambda j, i: (j, 0)),
            _full(gp.shape), _full(bp.shape), _full(p['W_e'].shape),
            _full(be.shape), _full(ge.shape), _full(bee.shape),
            _full(Wna.shape), _full(Wea.shape), _full(wda.shape),
            _full(ba.shape), _full(p['W_self'].shape), _full(gs.shape),
            _full(bs.shape),
        ],
        out_specs=[
            pl.BlockSpec((JB, Ds), lambda j, i: (j, 0)),
            pl.BlockSpec((JB, 9), lambda j, i: (j, 0)),
        ],
        out_shape=[
            jax.ShapeDtypeStruct((L, Ds), jnp.float32),
            jax.ShapeDtypeStruct((L, 9), jnp.float32),
        ],
        scratch_shapes=[
            pltpu.VMEM((JB, Ds), jnp.float32),
            pltpu.VMEM((JB, 9), jnp.float32),
        ],
        compiler_params=pltpu.CompilerParams(
            dimension_semantics=("parallel", "arbitrary")),
        interpret=interpret,
    )(pair3, mask, node, node, cac, car, xyz9,
      gp, bp, p['W_e'], be, ge, bee, Wna, Wea, wda, ba, p['W_self'], gs, bs)

    return state.reshape(B, L, Ds), xyzo.reshape(B, L, 3, 3)


# speedup vs baseline: 2.2529x; 2.2529x over previous
import functools

import jax
import jax.numpy as jnp
from jax import lax
from jax.experimental import pallas as pl
from jax.experimental.pallas import tpu as pltpu
from jax.experimental.pallas import tpu_sc as plsc

EPS = 1e-5
CAP = 80
DUMP = 512


def _ln(x, g, b):
    mu = jnp.mean(x, axis=-1, keepdims=True)
    var = jnp.mean((x - mu) ** 2, axis=-1, keepdims=True)
    return (x - mu) * jax.lax.rsqrt(var + EPS) * g + b


def _elu(x):
    return jnp.where(x > 0, x, jnp.exp(x) - 1.0)


def _node_kernel(msa_ref, seq_ref, gm_ref, bm_ref, Wq_ref, bq_ref, WkT_ref,
                 Wnm_ref, Wns_ref, bn_ref, gn_ref, bnn_ref, node_ref, *, scale):
    x = msa_ref[...]
    xn = _ln(x, gm_ref[...], bm_ref[...])
    q = (jnp.dot(xn[0], Wq_ref[...], preferred_element_type=jnp.float32)
         + bq_ref[...]) * scale
    qw = jnp.dot(q, WkT_ref[...], preferred_element_type=jnp.float32)
    logits = jnp.sum(xn * qw[None, :, :], axis=-1)
    mx = jnp.max(logits, axis=0, keepdims=True)
    e = jnp.exp(logits - mx)
    att = e / jnp.sum(e, axis=0, keepdims=True)
    ws = jnp.sum(xn * att[:, :, None], axis=0)
    pre = (jnp.dot(ws, Wnm_ref[...], preferred_element_type=jnp.float32)
           + jnp.dot(seq_ref[...], Wns_ref[...], preferred_element_type=jnp.float32)
           + bn_ref[...])
    node_ref[...] = _ln(_elu(pre), gn_ref[...], bnn_ref[...])


def _mask_kernel(cac_ref, car_ref, aac_ref, aar_ref, jdxt_ref, gidxt_ref,
                 km_ref, *, L, K, kmin):
    dx = [car_ref[c:c + 1, :] - cac_ref[:, c:c + 1] for c in range(3)]
    pd2 = dx[0] * dx[0] + dx[1] * dx[1] + dx[2] * dx[2]
    ri = jax.lax.broadcasted_iota(jnp.int32, (L, L), 0)
    ci = jax.lax.broadcasted_iota(jnp.int32, (L, L), 1)
    diag = ri == ci
    pdist = jnp.sqrt(pd2 + 1e-12) + jnp.where(diag, 1000.0, 0.0)
    bits = jax.lax.bitcast_convert_type(pdist, jnp.int32)

    def body(_, carry):
        lo, hi = carry
        mid = lo + jax.lax.shift_right_logical(hi - lo, 1)
        cnt = jnp.sum((bits <= mid).astype(jnp.int32), axis=0, keepdims=True)
        ge = cnt >= K
        return jnp.where(ge, lo, mid), jnp.where(ge, mid, hi)

    lo0 = jnp.full((1, L), -1, jnp.int32)
    hi0 = jnp.full((1, L), 0x7F7FFFFF, jnp.int32)
    _, t = jax.lax.fori_loop(0, 31, body, (lo0, hi0))

    eqt_t = bits == t
    c_lt = jnp.sum((bits < t).astype(jnp.int32), axis=0, keepdims=True)
    needed = K - c_lt

    def body2(_, carry):
        lo, hi = carry
        mid = lo + jax.lax.shift_right_logical(hi - lo, 1)
        cnt = jnp.sum(jnp.logical_and(eqt_t, ri <= mid).astype(jnp.int32),
                      axis=0, keepdims=True)
        ge = cnt >= needed
        return jnp.where(ge, lo, mid), jnp.where(ge, mid, hi)

    jlo0 = jnp.full((1, L), -1, jnp.int32)
    jhi0 = jnp.full((1, L), L - 1, jnp.int32)
    _, jt = jax.lax.fori_loop(0, 10, body2, (jlo0, jhi0))
    knn_t = jnp.logical_or(bits < t,
                           jnp.logical_and(bits == t, ri <= jt))
    aa_d = jnp.abs(aac_ref[...] - aar_ref[...])
    band = jnp.logical_and(aa_d < kmin, jnp.logical_not(diag))
    INF = jnp.int32(0x7F800000)
    km_ref[...] = jnp.where(jnp.logical_or(knn_t, band), bits, INF)

    irow = jax.lax.broadcasted_iota(jnp.int32, (1, L), 1)

    def extract(s, carry):
        km = km_ref[...]
        minv = jnp.min(km, axis=0, keepdims=True)
        idxs = jnp.min(jnp.where(km == minv, ri, L), axis=0,
                       keepdims=True)
        valid = minv < INF
        jdxt_ref[pl.ds(s, 1), :] = jnp.where(valid, idxs, DUMP)
        gidxt_ref[pl.ds(s, 1), :] = irow * L + jnp.where(valid, idxs, irow)
        km_ref[...] = jnp.where(ri == idxs, INF, km)
        return carry

    jax.lax.fori_loop(0, CAP, extract, 0)


def _sc_gather_body(gidx_hbm, jdx_hbm, pairflat_hbm, ca128_hbm,
                    edges_hbm, cap_hbm,
                    gidx_v, jdx_v, erow_v, cap_v, sem, *, L, rows_per_w):
    nc = 2
    wid = lax.axis_index("s") * nc + lax.axis_index("c")

    def row_body(r, carry):
        i = wid * rows_per_w + r
        pltpu.sync_copy(gidx_hbm.at[i], gidx_v)
        pltpu.sync_copy(jdx_hbm.at[i], jdx_v)
        pltpu.async_copy(pairflat_hbm.at[gidx_v], erow_v, sem).wait()
        pltpu.async_copy(ca128_hbm.at[jdx_v], cap_v, sem).wait()
        pltpu.sync_copy(erow_v, edges_hbm.at[pl.ds(i * CAP, CAP)])
        pltpu.sync_copy(cap_v, cap_hbm.at[pl.ds(i * CAP, CAP)])
        return carry

    lax.fori_loop(0, rows_per_w, row_body, jnp.int32(0))


def _sc_scatter_body(msg_hbm, jdxall_hbm, zeros_hbm, agg_hbm,
                     msg_v, jdx_v, shared, sem, *, L, rows_per_w):
    cid = lax.axis_index("c")
    sid = lax.axis_index("s")

    @pl.when(jnp.logical_and(cid == 0, sid == 0))
    def _():
        pltpu.sync_copy(zeros_hbm, shared)

    plsc.subcore_barrier()

    @pl.when(cid == 0)
    def _():
        def row_body(r, carry):
            i = sid * rows_per_w + r
            pltpu.sync_copy(msg_hbm.at[pl.ds(i * CAP, CAP)], msg_v)
            pltpu.sync_copy(jdxall_hbm.at[i], jdx_v)
            pltpu.sync_copy(msg_v, shared.at[jdx_v], add=True)
            return carry

        lax.fori_loop(0, rows_per_w, row_body, jnp.int32(0))

    plsc.subcore_barrier()

    @pl.when(jnp.logical_and(cid == 0, sid == 0))
    def _():
        pltpu.sync_copy(shared.at[pl.ds(0, L)], agg_hbm)


def _edge_kernel(edges_ref, cap_ref, node_ref, ca_ref,
                 gp_ref, bp_ref, We_ref, be_ref, ge_ref, bee_ref,
                 Wna_ref, Wea_ref, wda_ref, ba_ref, out_ref, *, nf, nsrc):
    RB = edges_ref.shape[0]
    x = edges_ref[...]
    xn = _ln(x, gp_ref[...], bp_ref[...])
    e0 = _elu(jnp.dot(xn, We_ref[...], preferred_element_type=jnp.float32)
              + be_ref[...])
    edge = _ln(e0, ge_ref[...], bee_ref[...])
    R = (lax.broadcasted_iota(jnp.int32, (RB, nsrc), 0) // CAP
         == lax.broadcasted_iota(jnp.int32, (RB, nsrc), 1)).astype(jnp.float32)
    nterm = jnp.dot(node_ref[...], Wna_ref[...], preferred_element_type=jnp.float32)
    cai = jnp.dot(R, ca_ref[...][:, :8], preferred_element_type=jnp.float32)
    dvec = cap_ref[...][:, :3] - cai[:, :3]
    dist = jnp.sqrt(dvec[:, :1] ** 2 + dvec[:, 1:2] ** 2 + dvec[:, 2:3] ** 2)
    pre = (jnp.dot(edge, Wea_ref[...], preferred_element_type=jnp.float32)
           + jnp.dot(R, nterm, preferred_element_type=jnp.float32)
           + dist * wda_ref[...] + ba_ref[...])
    m = _elu(pre[:, :32])
    coef = pre[:, 32:]
    vm = jnp.concatenate([coef * dvec[:, c:c + 1] for c in range(3)], axis=1)
    pad = jnp.zeros((RB, 128 - 32 - 9), jnp.float32)
    out_ref[...] = jnp.concatenate([m, vm, pad], axis=1)


def _final_kernel(agg_ref, node_ref, xyz9_ref, Wself_ref, gs_ref, bs_ref,
                  state_ref, xyzo_ref):
    agg = agg_ref[...][:, :32] + jnp.dot(node_ref[...], Wself_ref[...],
                                         preferred_element_type=jnp.float32)
    state_ref[...] = _ln(_elu(agg), gs_ref[...], bs_ref[...])
    d = agg_ref[...][:, 32:41]
    xin = xyz9_ref[...]
    da = [jnp.concatenate([d[:, a:a + 1], d[:, 3 + a:4 + a],
                           d[:, 6 + a:7 + a]], axis=1) for a in range(3)]
    ca_new = xin[:, 3:6] + da[1]
    xyzo_ref[...] = jnp.concatenate(
        [ca_new + da[0], ca_new, ca_new + da[2]], axis=1)


def _full(shape):
    return pl.BlockSpec(shape, lambda *args: (0,) * len(shape))


def kernel(xyz, msa, pair, seq_onehot, params, aa_idx, interpret=False):
    p = params
    B, L = xyz.shape[:2]
    N, Dm = msa.shape[1], msa.shape[3]
    Dp = pair.shape[3]
    Dn, Ds, NF = 64, 32, 35
    K, KMIN = 64, 9
    E = L * CAP

    msa3 = msa[0]
    pairflat = pair[0].reshape(L * L, Dp)
    seq = seq_onehot[0]
    ca = xyz[0, :, 1, :]
    cac = jnp.pad(ca, ((0, 0), (0, 5)))
    car = cac.T
    ca128 = jnp.zeros((2 * L, 128), jnp.float32).at[:L, :3].set(ca)
    aa = aa_idx[0].astype(jnp.int32)
    aac = aa.reshape(L, 1)
    aar = aa.reshape(1, L)
    xyz9 = xyz[0].reshape(L, 9)
    scale = float(Dm) ** -0.5

    gm = p['ln_msa_g'].reshape(1, 1, Dm)
    bm = p['ln_msa_b'].reshape(1, 1, Dm)
    bq = p['bq'].reshape(1, Dm)
    WkT = p['Wk'].T
    Wnm = p['W_n'][:Dm]
    Wns = p['W_n'][Dm:]
    bn = p['b_n'].reshape(1, Dn)
    gn = p['ln_node_g'].reshape(1, Dn)
    bnn = p['ln_node_b'].reshape(1, Dn)
    gp = p['ln_pair_g'].reshape(1, Dp)
    bp = p['ln_pair_b'].reshape(1, Dp)
    be = p['b_e'].reshape(1, Dn)
    ge = p['ln_edge_g'].reshape(1, Dn)
    bee = p['ln_edge_b'].reshape(1, Dn)
    W_all = jnp.concatenate([p['W_msg'], p['W_vec']], axis=1)
    Wna = W_all[:Dn]
    Wea = W_all[Dn:2 * Dn]
    wda = W_all[2 * Dn].reshape(1, NF)
    ba = jnp.concatenate([p['b_msg'], jnp.zeros((3,), jnp.float32)]).reshape(1, NF)
    gs = p['ln_state_g'].reshape(1, Ds)
    bs = p['ln_state_b'].reshape(1, Ds)

    LB = 64
    node = pl.pallas_call(
        functools.partial(_node_kernel, scale=scale),
        grid=(L // LB,),
        in_specs=[
            pl.BlockSpec((N, LB, Dm), lambda l: (0, l, 0)),
            pl.BlockSpec((LB, seq.shape[1]), lambda l: (l, 0)),
            _full(gm.shape), _full(bm.shape), _full(p['Wq'].shape),
            _full(bq.shape), _full(WkT.shape), _full(Wnm.shape),
            _full(Wns.shape), _full(bn.shape), _full(gn.shape),
            _full(bnn.shape),
        ],
        out_specs=pl.BlockSpec((LB, Dn), lambda l: (l, 0)),
        out_shape=jax.ShapeDtypeStruct((L, Dn), jnp.float32),
        interpret=interpret,
    )(msa3, seq, gm, bm, p['Wq'], bq, WkT, Wnm, Wns, bn, gn, bnn)

    jdxt, gidxt = pl.pallas_call(
        functools.partial(_mask_kernel, L=L, K=K, kmin=KMIN),
        out_shape=[jax.ShapeDtypeStruct((CAP, L), jnp.int32),
                   jax.ShapeDtypeStruct((CAP, L), jnp.int32)],
        scratch_shapes=[pltpu.VMEM((L, L), jnp.int32)],
        interpret=interpret,
    )(cac, car, aac, aar)
    jdx = jdxt.T
    gidx = gidxt.T

    mesh = plsc.VectorSubcoreMesh(core_axis_name="c", subcore_axis_name="s")
    edges, cap = pl.kernel(
        functools.partial(_sc_gather_body, L=L, rows_per_w=L // 32),
        out_type=[
            jax.ShapeDtypeStruct((E, Dp), jnp.float32),
            jax.ShapeDtypeStruct((E, 128), jnp.float32),
        ],
        mesh=mesh,
        scratch_types=[
            pltpu.VMEM((CAP,), jnp.int32),
            pltpu.VMEM((CAP,), jnp.int32),
            pltpu.VMEM((CAP, Dp), jnp.float32),
            pltpu.VMEM((CAP, 128), jnp.float32),
            pltpu.SemaphoreType.DMA,
        ],
    )(gidx, jdx, pairflat, ca128)

    NSRC = 16
    RB = NSRC * CAP
    msg = pl.pallas_call(
        functools.partial(_edge_kernel, nf=NF, nsrc=NSRC),
        grid=(L // NSRC,),
        in_specs=[
            pl.BlockSpec((RB, Dp), lambda g: (g, 0)),
            pl.BlockSpec((RB, 128), lambda g: (g, 0)),
            pl.BlockSpec((NSRC, Dn), lambda g: (g, 0)),
            pl.BlockSpec((NSRC, 128), lambda g: (g, 0)),
            _full(gp.shape), _full(bp.shape), _full(p['W_e'].shape),
            _full(be.shape), _full(ge.shape), _full(bee.shape),
            _full(Wna.shape), _full(Wea.shape), _full(wda.shape),
            _full(ba.shape),
        ],
        out_specs=pl.BlockSpec((RB, 128), lambda g: (g, 0)),
        out_shape=jax.ShapeDtypeStruct((E, 128), jnp.float32),
        interpret=interpret,
    )(edges, cap, node, ca128, gp, bp, p['W_e'], be, ge, bee, Wna, Wea, wda, ba)

    zeros_acc = jnp.zeros((L + 8, 128), jnp.float32)
    agg = pl.kernel(
        functools.partial(_sc_scatter_body, L=L, rows_per_w=L // 16),
        out_type=jax.ShapeDtypeStruct((L, 128), jnp.float32),
        mesh=mesh,
        scratch_types=[
            pltpu.VMEM((CAP, 128), jnp.float32),
            pltpu.VMEM((CAP,), jnp.int32),
            pltpu.VMEM_SHARED((L + 8, 128), jnp.float32),
            pltpu.SemaphoreType.DMA,
        ],
    )(msg, jdx, zeros_acc)

    state, xyzo = pl.pallas_call(
        _final_kernel,
        in_specs=[_full((L, 128)), _full((L, Dn)), _full((L, 9)),
                  _full(p['W_self'].shape), _full(gs.shape), _full(bs.shape)],
        out_specs=[pl.BlockSpec((L, Ds), lambda: (0, 0)),
                   pl.BlockSpec((L, 9), lambda: (0, 0))],
        out_shape=[
            jax.ShapeDtypeStruct((L, Ds), jnp.float32),
            jax.ShapeDtypeStruct((L, 9), jnp.float32),
        ],
        interpret=interpret,
    )(agg, node, xyz9, p['W_self'], gs, bs)

    return state.reshape(B, L, Ds), xyzo.reshape(B, L, 3, 3)


# speedup vs baseline: 2.9207x; 1.2964x over previous
import functools

import jax
import jax.numpy as jnp
from jax import lax
from jax.experimental import pallas as pl
from jax.experimental.pallas import tpu as pltpu
from jax.experimental.pallas import tpu_sc as plsc

EPS = 1e-5
CAP = 80
DUMP = 512


def _ln(x, g, b):
    mu = jnp.mean(x, axis=-1, keepdims=True)
    var = jnp.mean((x - mu) ** 2, axis=-1, keepdims=True)
    return (x - mu) * jax.lax.rsqrt(var + EPS) * g + b


def _elu(x):
    return jnp.where(x > 0, x, jnp.exp(x) - 1.0)


def _node_kernel(msa_ref, seq_ref, gm_ref, bm_ref, Wq_ref, bq_ref, WkT_ref,
                 Wnm_ref, Wns_ref, bn_ref, gn_ref, bnn_ref, node_ref, *, scale):
    x = msa_ref[...]
    xn = _ln(x, gm_ref[...], bm_ref[...])
    q = (jnp.dot(xn[0], Wq_ref[...], preferred_element_type=jnp.float32)
         + bq_ref[...]) * scale
    qw = jnp.dot(q, WkT_ref[...], preferred_element_type=jnp.float32)
    logits = jnp.sum(xn * qw[None, :, :], axis=-1)
    mx = jnp.max(logits, axis=0, keepdims=True)
    e = jnp.exp(logits - mx)
    att = e / jnp.sum(e, axis=0, keepdims=True)
    ws = jnp.sum(xn * att[:, :, None], axis=0)
    pre = (jnp.dot(ws, Wnm_ref[...], preferred_element_type=jnp.float32)
           + jnp.dot(seq_ref[...], Wns_ref[...], preferred_element_type=jnp.float32)
           + bn_ref[...])
    node_ref[...] = _ln(_elu(pre), gn_ref[...], bnn_ref[...])


def _mask_kernel(cac_ref, car_ref, aac_ref, aar_ref, jdxt_ref, gidxt_ref,
                 dist_ref, dv0_ref, dv1_ref, dv2_ref, km_ref, *, L, K, kmin):
    dx = [car_ref[c:c + 1, :] - cac_ref[:, c:c + 1] for c in range(3)]
    pd2 = dx[0] * dx[0] + dx[1] * dx[1] + dx[2] * dx[2]
    ri = jax.lax.broadcasted_iota(jnp.int32, (L, L), 0)
    ci = jax.lax.broadcasted_iota(jnp.int32, (L, L), 1)
    diag = ri == ci
    pdist = jnp.sqrt(pd2 + 1e-12) + jnp.where(diag, 1000.0, 0.0)
    bits = jax.lax.bitcast_convert_type(pdist, jnp.int32)

    def body(_, carry):
        lo, hi = carry
        mid = lo + jax.lax.shift_right_logical(hi - lo, 1)
        cnt = jnp.sum((bits <= mid).astype(jnp.int32), axis=0, keepdims=True)
        ge = cnt >= K
        return jnp.where(ge, lo, mid), jnp.where(ge, mid, hi)

    lo0 = jnp.full((1, L), -1, jnp.int32)
    hi0 = jnp.full((1, L), 0x7F7FFFFF, jnp.int32)
    _, t = jax.lax.fori_loop(0, 31, body, (lo0, hi0))

    eqt_t = bits == t
    c_lt = jnp.sum((bits < t).astype(jnp.int32), axis=0, keepdims=True)
    needed = K - c_lt

    def body2(_, carry):
        lo, hi = carry
        mid = lo + jax.lax.shift_right_logical(hi - lo, 1)
        cnt = jnp.sum(jnp.logical_and(eqt_t, ri <= mid).astype(jnp.int32),
                      axis=0, keepdims=True)
        ge = cnt >= needed
        return jnp.where(ge, lo, mid), jnp.where(ge, mid, hi)

    jlo0 = jnp.full((1, L), -1, jnp.int32)
    jhi0 = jnp.full((1, L), L - 1, jnp.int32)
    _, jt = jax.lax.fori_loop(0, 10, body2, (jlo0, jhi0))
    knn_t = jnp.logical_or(bits < t,
                           jnp.logical_and(bits == t, ri <= jt))
    aa_d = jnp.abs(aac_ref[...] - aar_ref[...])
    band = jnp.logical_and(aa_d < kmin, jnp.logical_not(diag))
    INF = jnp.int32(0x7F800000)
    km_ref[...] = jnp.where(jnp.logical_or(knn_t, band), bits, INF)

    irow = jax.lax.broadcasted_iota(jnp.int32, (1, L), 1)

    carf = car_ref[...]

    def extract(s, carry):
        km = km_ref[...]
        minv = jnp.min(km, axis=0, keepdims=True)
        idxs = jnp.min(jnp.where(km == minv, ri, L), axis=0,
                       keepdims=True)
        valid = minv < INF
        jdxt_ref[pl.ds(s, 1), :] = jnp.where(valid, idxs, DUMP)
        gidxt_ref[pl.ds(s, 1), :] = irow * L + jnp.where(valid, idxs, irow)
        distf = jax.lax.bitcast_convert_type(minv, jnp.float32)
        dist_ref[pl.ds(s, 1), :] = jnp.where(valid, distf, 0.0)
        onehot = ri == idxs
        ohf = jnp.where(onehot, 1.0, 0.0)
        caj = jnp.dot(carf, ohf, preferred_element_type=jnp.float32)
        dv = caj - carf
        dv0_ref[pl.ds(s, 1), :] = jnp.where(valid, dv[0:1, :], 0.0)
        dv1_ref[pl.ds(s, 1), :] = jnp.where(valid, dv[1:2, :], 0.0)
        dv2_ref[pl.ds(s, 1), :] = jnp.where(valid, dv[2:3, :], 0.0)
        km_ref[...] = jnp.where(onehot, INF, km)
        return carry

    jax.lax.fori_loop(0, CAP, extract, 0)


def _sc_gather_body(gidx_hbm, pairflat_hbm, edges_hbm,
                    gidx_v, gidx2_v, erow_v, erow2_v, sem, *, L, rows_per_w):
    nc = 2
    wid = lax.axis_index("s") * nc + lax.axis_index("c")

    def row_pair(r2, carry):
        i0 = wid * rows_per_w + r2 * 2
        pltpu.sync_copy(gidx_hbm.at[i0], gidx_v)
        cp0 = pltpu.async_copy(pairflat_hbm.at[gidx_v], erow_v, sem)
        pltpu.sync_copy(gidx_hbm.at[i0 + 1], gidx2_v)
        cp0.wait()
        cp1 = pltpu.async_copy(pairflat_hbm.at[gidx2_v], erow2_v, sem)
        pltpu.sync_copy(erow_v, edges_hbm.at[pl.ds(i0 * CAP, CAP)])
        cp1.wait()
        pltpu.sync_copy(erow2_v, edges_hbm.at[pl.ds((i0 + 1) * CAP, CAP)])
        return carry

    lax.fori_loop(0, rows_per_w // 2, row_pair, jnp.int32(0))


def _sc_scatter_body(msg_hbm, jdxall_hbm, zeros_hbm, agg_hbm,
                     msg_v, jdx_v, shared, sem, *, L, rows_per_w):
    cid = lax.axis_index("c")
    sid = lax.axis_index("s")
    wid = sid * 2 + cid

    @pl.when(sid == 0)
    def _():
        pltpu.sync_copy(zeros_hbm, shared)

    plsc.subcore_barrier()

    def row_body(r, carry):
        i = wid * rows_per_w + r
        pltpu.sync_copy(msg_hbm.at[pl.ds(i * CAP, CAP)], msg_v)
        pltpu.sync_copy(jdxall_hbm.at[i], jdx_v)
        pltpu.sync_copy(msg_v, shared.at[jdx_v], add=True)
        return carry

    lax.fori_loop(0, rows_per_w, row_body, jnp.int32(0))

    plsc.subcore_barrier()

    @pl.when(sid == 0)
    def _():
        pltpu.sync_copy(shared.at[pl.ds(0, L)], agg_hbm.at[cid])


def _edge_kernel(edges_ref, feats_ref, node_ref,
                 gp_ref, bp_ref, We_ref, be_ref, ge_ref, bee_ref,
                 Wna_ref, Wea_ref, wda_ref, ba_ref, out_ref, *, nf, nsrc):
    RB = edges_ref.shape[0]
    x = edges_ref[...]
    xn = _ln(x, gp_ref[...], bp_ref[...])
    e0 = _elu(jnp.dot(xn, We_ref[...], preferred_element_type=jnp.float32)
              + be_ref[...])
    edge = _ln(e0, ge_ref[...], bee_ref[...])
    R = (lax.broadcasted_iota(jnp.int32, (RB, nsrc), 0) // CAP
         == lax.broadcasted_iota(jnp.int32, (RB, nsrc), 1)).astype(jnp.float32)
    nterm = jnp.dot(node_ref[...], Wna_ref[...], preferred_element_type=jnp.float32)
    f = feats_ref[...]
    dist = f[:, 3:4]
    pre = (jnp.dot(edge, Wea_ref[...], preferred_element_type=jnp.float32)
           + jnp.dot(R, nterm, preferred_element_type=jnp.float32)
           + dist * wda_ref[...] + ba_ref[...])
    m = _elu(pre[:, :32])
    coef = pre[:, 32:]
    vm = jnp.concatenate([coef * f[:, c:c + 1] for c in range(3)], axis=1)
    pad = jnp.zeros((RB, 128 - 32 - 9), jnp.float32)
    out_ref[...] = jnp.concatenate([m, vm, pad], axis=1)


def _final_kernel(agg_ref, node_ref, xyz9_ref, Wself_ref, gs_ref, bs_ref,
                  state_ref, xyzo_ref):
    a2 = agg_ref[...][0] + agg_ref[...][1]
    agg = a2[:, :32] + jnp.dot(node_ref[...], Wself_ref[...],
                               preferred_element_type=jnp.float32)
    state_ref[...] = _ln(_elu(agg), gs_ref[...], bs_ref[...])
    d = a2[:, 32:41]
    xin = xyz9_ref[...]
    da = [jnp.concatenate([d[:, a:a + 1], d[:, 3 + a:4 + a],
                           d[:, 6 + a:7 + a]], axis=1) for a in range(3)]
    ca_new = xin[:, 3:6] + da[1]
    xyzo_ref[...] = jnp.concatenate(
        [ca_new + da[0], ca_new, ca_new + da[2]], axis=1)


def _full(shape):
    return pl.BlockSpec(shape, lambda *args: (0,) * len(shape))


def kernel(xyz, msa, pair, seq_onehot, params, aa_idx, interpret=False):
    p = params
    B, L = xyz.shape[:2]
    N, Dm = msa.shape[1], msa.shape[3]
    Dp = pair.shape[3]
    Dn, Ds, NF = 64, 32, 35
    K, KMIN = 64, 9
    E = L * CAP

    msa3 = msa[0]
    pairflat = pair[0].reshape(L * L, Dp)
    seq = seq_onehot[0]
    ca = xyz[0, :, 1, :]
    cac = jnp.pad(ca, ((0, 0), (0, 5)))
    car = cac.T
    aa = aa_idx[0].astype(jnp.int32)
    aac = aa.reshape(L, 1)
    aar = aa.reshape(1, L)
    xyz9 = xyz[0].reshape(L, 9)
    scale = float(Dm) ** -0.5

    gm = p['ln_msa_g'].reshape(1, 1, Dm)
    bm = p['ln_msa_b'].reshape(1, 1, Dm)
    bq = p['bq'].reshape(1, Dm)
    WkT = p['Wk'].T
    Wnm = p['W_n'][:Dm]
    Wns = p['W_n'][Dm:]
    bn = p['b_n'].reshape(1, Dn)
    gn = p['ln_node_g'].reshape(1, Dn)
    bnn = p['ln_node_b'].reshape(1, Dn)
    gp = p['ln_pair_g'].reshape(1, Dp)
    bp = p['ln_pair_b'].reshape(1, Dp)
    be = p['b_e'].reshape(1, Dn)
    ge = p['ln_edge_g'].reshape(1, Dn)
    bee = p['ln_edge_b'].reshape(1, Dn)
    W_all = jnp.concatenate([p['W_msg'], p['W_vec']], axis=1)
    Wna = W_all[:Dn]
    Wea = W_all[Dn:2 * Dn]
    wda = W_all[2 * Dn].reshape(1, NF)
    ba = jnp.concatenate([p['b_msg'], jnp.zeros((3,), jnp.float32)]).reshape(1, NF)
    gs = p['ln_state_g'].reshape(1, Ds)
    bs = p['ln_state_b'].reshape(1, Ds)

    LB = 64
    node = pl.pallas_call(
        functools.partial(_node_kernel, scale=scale),
        grid=(L // LB,),
        in_specs=[
            pl.BlockSpec((N, LB, Dm), lambda l: (0, l, 0)),
            pl.BlockSpec((LB, seq.shape[1]), lambda l: (l, 0)),
            _full(gm.shape), _full(bm.shape), _full(p['Wq'].shape),
            _full(bq.shape), _full(WkT.shape), _full(Wnm.shape),
            _full(Wns.shape), _full(bn.shape), _full(gn.shape),
            _full(bnn.shape),
        ],
        out_specs=pl.BlockSpec((LB, Dn), lambda l: (l, 0)),
        out_shape=jax.ShapeDtypeStruct((L, Dn), jnp.float32),
        interpret=interpret,
    )(msa3, seq, gm, bm, p['Wq'], bq, WkT, Wnm, Wns, bn, gn, bnn)

    jdxt, gidxt, distt, dv0t, dv1t, dv2t = pl.pallas_call(
        functools.partial(_mask_kernel, L=L, K=K, kmin=KMIN),
        out_shape=[jax.ShapeDtypeStruct((CAP, L), jnp.int32),
                   jax.ShapeDtypeStruct((CAP, L), jnp.int32),
                   jax.ShapeDtypeStruct((CAP, L), jnp.float32),
                   jax.ShapeDtypeStruct((CAP, L), jnp.float32),
                   jax.ShapeDtypeStruct((CAP, L), jnp.float32),
                   jax.ShapeDtypeStruct((CAP, L), jnp.float32)],
        scratch_shapes=[pltpu.VMEM((L, L), jnp.int32)],
        interpret=interpret,
    )(cac, car, aac, aar)
    jdx = jdxt.T
    gidx = gidxt.T
    feats = jnp.concatenate(
        [dv0t.T.reshape(E, 1), dv1t.T.reshape(E, 1), dv2t.T.reshape(E, 1),
         distt.T.reshape(E, 1), jnp.zeros((E, 4), jnp.float32)], axis=1)

    mesh = plsc.VectorSubcoreMesh(core_axis_name="c", subcore_axis_name="s")
    edges = pl.kernel(
        functools.partial(_sc_gather_body, L=L, rows_per_w=L // 32),
        out_type=jax.ShapeDtypeStruct((E, Dp), jnp.float32),
        mesh=mesh,
        scratch_types=[
            pltpu.VMEM((CAP,), jnp.int32),
            pltpu.VMEM((CAP,), jnp.int32),
            pltpu.VMEM((CAP, Dp), jnp.float32),
            pltpu.VMEM((CAP, Dp), jnp.float32),
            pltpu.SemaphoreType.DMA,
        ],
    )(gidx, pairflat)

    NSRC = 16
    RB = NSRC * CAP
    msg = pl.pallas_call(
        functools.partial(_edge_kernel, nf=NF, nsrc=NSRC),
        grid=(L // NSRC,),
        in_specs=[
            pl.BlockSpec((RB, Dp), lambda g: (g, 0)),
            pl.BlockSpec((RB, 8), lambda g: (g, 0)),
            pl.BlockSpec((NSRC, Dn), lambda g: (g, 0)),
            _full(gp.shape), _full(bp.shape), _full(p['W_e'].shape),
            _full(be.shape), _full(ge.shape), _full(bee.shape),
            _full(Wna.shape), _full(Wea.shape), _full(wda.shape),
            _full(ba.shape),
        ],
        out_specs=pl.BlockSpec((RB, 128), lambda g: (g, 0)),
        out_shape=jax.ShapeDtypeStruct((E, 128), jnp.float32),
        interpret=interpret,
    )(edges, feats, node, gp, bp, p['W_e'], be, ge, bee, Wna, Wea, wda, ba)

    zeros_acc = jnp.zeros((L + 8, 128), jnp.float32)
    agg = pl.kernel(
        functools.partial(_sc_scatter_body, L=L, rows_per_w=L // 32),
        out_type=jax.ShapeDtypeStruct((2, L, 128), jnp.float32),
        mesh=mesh,
        scratch_types=[
            pltpu.VMEM((CAP, 128), jnp.float32),
            pltpu.VMEM((CAP,), jnp.int32),
            pltpu.VMEM_SHARED((L + 8, 128), jnp.float32),
            pltpu.SemaphoreType.DMA,
        ],
    )(msg, jdx, zeros_acc)

    state, xyzo = pl.pallas_call(
        _final_kernel,
        in_specs=[_full((2, L, 128)), _full((L, Dn)), _full((L, 9)),
                  _full(p['W_self'].shape), _full(gs.shape), _full(bs.shape)],
        out_specs=[pl.BlockSpec((L, Ds), lambda: (0, 0)),
                   pl.BlockSpec((L, 9), lambda: (0, 0))],
        out_shape=[
            jax.ShapeDtypeStruct((L, Ds), jnp.float32),
            jax.ShapeDtypeStruct((L, 9), jnp.float32),
        ],
        interpret=interpret,
    )(agg, node, xyz9, p['W_self'], gs, bs)

    return state.reshape(B, L, Ds), xyzo.reshape(B, L, 3, 3)
